# baseline (speedup 1.0000x reference)
import functools

import jax
import jax.numpy as jnp
from jax import lax
from jax.experimental import pallas as pl
from jax.experimental.pallas import tpu as pltpu
from jax.experimental.pallas import tpu_sc as plsc

B = 16384
D = 64
NC = 2
NS = 16
NW = NC * NS
BPW = B // NW
CHUNK = 128
NCHUNK = BPW // CHUNK


def _sc_gather(u_ids, i_ids, user_weight, item_weight):
    mesh = plsc.VectorSubcoreMesh(
        core_axis_name="c", subcore_axis_name="s", num_cores=NC, num_subcores=NS
    )
    row_t = jax.ShapeDtypeStruct((B, D), jnp.float32)

    @functools.partial(
        pl.kernel,
        out_type=[row_t, row_t],
        mesh=mesh,
        scratch_types=[
            pltpu.VMEM((NCHUNK, CHUNK), jnp.int32),
            pltpu.VMEM((NCHUNK, CHUNK), jnp.int32),
            pltpu.VMEM((BPW, D), jnp.float32),
            pltpu.VMEM((BPW, D), jnp.float32),
            pltpu.SemaphoreType.DMA,
        ],
        compiler_params=pltpu.CompilerParams(use_tc_tiling_on_sc=False),
    )
    def k(u_tbl, i_tbl, uid_hbm, iid_hbm, u_out, i_out, uid_v, iid_v, u_rows, i_rows, sem):
        wid = lax.axis_index("s") * NC + lax.axis_index("c")
        base = wid * BPW
        pltpu.sync_copy(uid_hbm.at[wid], uid_v)
        pltpu.sync_copy(iid_hbm.at[wid], iid_v)
        copies = []
        for j in range(NCHUNK):
            dst = pl.ds(j * CHUNK, CHUNK)
            copies.append(pltpu.async_copy(u_tbl.at[uid_v.at[j]], u_rows.at[dst], sem))
            copies.append(pltpu.async_copy(i_tbl.at[iid_v.at[j]], i_rows.at[dst], sem))
        for c in copies:
            c.wait()
        pltpu.sync_copy(u_rows, u_out.at[pl.ds(base, BPW)])
        pltpu.sync_copy(i_rows, i_out.at[pl.ds(base, BPW)])

    uid3 = u_ids.reshape(NW, NCHUNK, CHUNK)
    iid3 = i_ids.reshape(NW, NCHUNK, CHUNK)
    return k(user_weight, item_weight, uid3, iid3)


def _tc_dot_body(u_ref, i_ref, o_ref):
    s = jnp.sum(u_ref[...] * i_ref[...], axis=1)
    o_ref[...] = s.reshape(o_ref.shape)


def _tc_dot(u_e, i_e):
    rows_per_blk = 2048
    grid = (B // rows_per_blk,)
    out = pl.pallas_call(
        _tc_dot_body,
        grid=grid,
        in_specs=[
            pl.BlockSpec((rows_per_blk, D), lambda i: (i, 0)),
            pl.BlockSpec((rows_per_blk, D), lambda i: (i, 0)),
        ],
        out_specs=pl.BlockSpec((rows_per_blk // 128, 128), lambda i: (i, 0)),
        out_shape=jax.ShapeDtypeStruct((B // 128, 128), jnp.float32),
    )(u_e, i_e)
    return out.reshape(B)


def kernel(u_ids, i_ids, user_weight, item_weight):
    u_e, i_e = _sc_gather(u_ids, i_ids, user_weight, item_weight)
    return _tc_dot(u_e, i_e)
